# Initial kernel scaffold; baseline (speedup 1.0000x reference)
import jax
import jax.numpy as jnp
from jax.experimental import pallas as pl


def kernel(feats, pseudo_lbls, src_prototype, src_prototype_cov):
    raise NotImplementedError("write your pallas kernel here")


# speedup vs baseline: 13.5940x; 13.5940x over previous
import jax
import jax.numpy as jnp
from jax.experimental import pallas as pl
from jax.experimental.pallas import tpu as pltpu

N = 320000
D = 128
C = 128
BLK = 16000
GRID = N // BLK


def _seg_loss_kernel(lbl_ref, feats_ref, proto_ref, cov_ref, out_ref,
                     acc_ref, cnt_ref):
    i = pl.program_id(0)
    labels = jnp.reshape(lbl_ref[...], (BLK, 1))
    oh = (labels == jax.lax.broadcasted_iota(jnp.int32, (BLK, C), 1)
          ).astype(jnp.float32)
    feats = feats_ref[...]
    partial = jax.lax.dot_general(
        oh, feats, (((0,), (0,)), ((), ())),
        preferred_element_type=jnp.float32)
    pcnt = jnp.sum(oh, axis=0, keepdims=True)

    @pl.when(i == 0)
    def _init():
        acc_ref[...] = partial
        cnt_ref[...] = pcnt

    @pl.when(i > 0)
    def _accum():
        acc_ref[...] += partial
        cnt_ref[...] += pcnt

    @pl.when(i == GRID - 1)
    def _epilogue():
        counts = cnt_ref[0, :]
        sums = acc_ref[...]
        means = sums / jnp.maximum(counts, 1.0)[:, None]
        present = (counts > 0.0).astype(jnp.float32)
        per_elem = (means - proto_ref[...]) ** 2 / (cov_ref[...] + 1e-6)
        per_elem = per_elem * present[:, None]
        loss = jnp.sum(per_elem) / (jnp.sum(present) * D)
        out_ref[...] = jnp.reshape(loss, (1, 1))


def kernel(feats, pseudo_lbls, src_prototype, src_prototype_cov):
    lbls3 = jnp.reshape(pseudo_lbls, (GRID, 1, BLK))
    out = pl.pallas_call(
        _seg_loss_kernel,
        grid=(GRID,),
        in_specs=[
            pl.BlockSpec((1, 1, BLK), lambda i: (i, 0, 0)),
            pl.BlockSpec((BLK, D), lambda i: (i, 0)),
            pl.BlockSpec((C, D), lambda i: (0, 0)),
            pl.BlockSpec((C, D), lambda i: (0, 0)),
        ],
        out_specs=pl.BlockSpec((1, 1), lambda i: (0, 0)),
        out_shape=jax.ShapeDtypeStruct((1, 1), jnp.float32),
        scratch_shapes=[
            pltpu.VMEM((C, D), jnp.float32),
            pltpu.VMEM((1, C), jnp.float32),
        ],
    )(lbls3, feats, src_prototype, src_prototype_cov)
    return out[0, 0]
